# baseline (speedup 1.0000x reference)
import jax
import jax.numpy as jnp
from jax import lax
from jax.experimental import pallas as pl
from jax.experimental.pallas import tpu as pltpu
from jax.experimental.pallas import tpu_sc as plsc

_VOCAB = 1000000
_HIDDEN = 64
_MAX_LEN = 200
_BATCH = 1024
_SEQ = 200

_NC, _NS = 2, 16
_NW = _NC * _NS
_TOTAL = _BATCH * _SEQ
_PER_W = _TOTAL // _NW
_CHUNK = 800
_NCHUNK = _PER_W // _CHUNK
_NREP = _CHUNK // _MAX_LEN


def _body(idx_hbm, tok_hbm, pos_hbm, out_hbm,
          idx0, idx1, rows0, rows1,
          s_idx0, s_idx1, s_pos0, s_pos1, s_g0, s_g1, s_o0, s_o1):
    wid = lax.axis_index("s") * _NC + lax.axis_index("c")
    base = wid * _PER_W

    idx_v = (idx0, idx1)
    rows_v = (rows0, rows1)
    s_idx = (s_idx0, s_idx1)
    s_pos = (s_pos0, s_pos1)
    s_g = (s_g0, s_g1)
    s_o = (s_o0, s_o1)

    gather_d = [None, None]
    out_d = [None, None]

    for i in range(_NCHUNK):
        b = i % 2
        off = base + i * _CHUNK
        if i >= 2:
            out_d[b].wait()
        di = pltpu.async_copy(idx_hbm.at[pl.ds(off, _CHUNK)], idx_v[b],
                              s_idx[b])
        dps = [pltpu.async_copy(
                   pos_hbm, rows_v[b].at[pl.ds(p * _MAX_LEN, _MAX_LEN)],
                   s_pos[b])
               for p in range(_NREP)]
        di.wait()
        for d in dps:
            d.wait()
        gather_d[b] = pltpu.async_copy(tok_hbm.at[idx_v[b]], rows_v[b],
                                       s_g[b], add=True)
        if i >= 1:
            pb = 1 - b
            gather_d[pb].wait()
            out_d[pb] = pltpu.async_copy(
                rows_v[pb],
                out_hbm.at[pl.ds(base + (i - 1) * _CHUNK, _CHUNK)],
                s_o[pb])

    last = (_NCHUNK - 1) % 2
    gather_d[last].wait()
    out_d[last] = pltpu.async_copy(
        rows_v[last],
        out_hbm.at[pl.ds(base + (_NCHUNK - 1) * _CHUNK, _CHUNK)],
        s_o[last])
    out_d[last].wait()
    out_d[1 - last].wait()


@jax.jit
def _embed(x_flat, token_table, position_table):
    mesh = plsc.VectorSubcoreMesh(core_axis_name="c", subcore_axis_name="s")
    return pl.kernel(
        _body,
        out_type=jax.ShapeDtypeStruct((_TOTAL, _HIDDEN), jnp.float32),
        mesh=mesh,
        scratch_types=[
            pltpu.VMEM((_CHUNK,), jnp.int32),
            pltpu.VMEM((_CHUNK,), jnp.int32),
            pltpu.VMEM((_CHUNK, _HIDDEN), jnp.float32),
            pltpu.VMEM((_CHUNK, _HIDDEN), jnp.float32),
            pltpu.SemaphoreType.DMA,
            pltpu.SemaphoreType.DMA,
            pltpu.SemaphoreType.DMA,
            pltpu.SemaphoreType.DMA,
            pltpu.SemaphoreType.DMA,
            pltpu.SemaphoreType.DMA,
            pltpu.SemaphoreType.DMA,
            pltpu.SemaphoreType.DMA,
        ],
        compiler_params=pltpu.CompilerParams(use_tc_tiling_on_sc=False),
    )(x_flat, token_table, position_table)


def kernel(x, token_table, position_table):
    x_flat = x.reshape(-1).astype(jnp.int32)
    out = _embed(x_flat, token_table, position_table)
    return out.reshape(_BATCH, _SEQ, _HIDDEN)


# speedup vs baseline: 1.1403x; 1.1403x over previous
import jax
import jax.numpy as jnp
from jax import lax
from jax.experimental import pallas as pl
from jax.experimental.pallas import tpu as pltpu
from jax.experimental.pallas import tpu_sc as plsc

_VOCAB = 1000000
_HIDDEN = 64
_MAX_LEN = 200
_BATCH = 1024
_SEQ = 200

_NC, _NS = 2, 16
_NW = _NC * _NS
_TOTAL = _BATCH * _SEQ
_PER_W = _TOTAL // _NW
_CHUNK = 800
_NCHUNK = _PER_W // _CHUNK
_NREP = _CHUNK // _MAX_LEN


def _body(idx_hbm, tok_hbm, pos_hbm, out_hbm,
          idx0, idx1, rows0, rows1,
          s_idx0, s_idx1, s_pos0, s_pos1, s_g0, s_g1, s_o0, s_o1):
    wid = lax.axis_index("s") * _NC + lax.axis_index("c")
    base = wid * _PER_W

    idx_v = (idx0, idx1)
    rows_v = (rows0, rows1)
    s_idx = (s_idx0, s_idx1)
    s_pos = (s_pos0, s_pos1)
    s_g = (s_g0, s_g1)
    s_o = (s_o0, s_o1)

    gather_d = [None, None]
    out_d = [None, None]

    for i in range(_NCHUNK):
        b = i % 2
        off = base + i * _CHUNK
        if i >= 2:
            out_d[b].wait()
        di = pltpu.async_copy(idx_hbm.at[pl.ds(off, _CHUNK)], idx_v[b],
                              s_idx[b])
        di.wait()
        _SUB = _CHUNK // 4
        gather_d[b] = [
            pltpu.async_copy(
                tok_hbm.at[idx_v[b].at[pl.ds(k * _SUB, _SUB)]],
                rows_v[b].at[pl.ds(k * _SUB, _SUB)],
                s_g[b])
            for k in range(4)]
        if i >= 1:
            pb = 1 - b
            for g in gather_d[pb]:
                g.wait()
            out_d[pb] = pltpu.async_copy(
                rows_v[pb],
                out_hbm.at[pl.ds(base + (i - 1) * _CHUNK, _CHUNK)],
                s_o[pb])

    last = (_NCHUNK - 1) % 2
    for g in gather_d[last]:
        g.wait()
    out_d[last] = pltpu.async_copy(
        rows_v[last],
        out_hbm.at[pl.ds(base + (_NCHUNK - 1) * _CHUNK, _CHUNK)],
        s_o[last])
    out_d[last].wait()
    out_d[1 - last].wait()


@jax.jit
def _embed(x_flat, token_table, position_table):
    mesh = plsc.VectorSubcoreMesh(core_axis_name="c", subcore_axis_name="s")
    return pl.kernel(
        _body,
        out_type=jax.ShapeDtypeStruct((_TOTAL, _HIDDEN), jnp.float32),
        mesh=mesh,
        scratch_types=[
            pltpu.VMEM((_CHUNK,), jnp.int32),
            pltpu.VMEM((_CHUNK,), jnp.int32),
            pltpu.VMEM((_CHUNK, _HIDDEN), jnp.float32),
            pltpu.VMEM((_CHUNK, _HIDDEN), jnp.float32),
            pltpu.SemaphoreType.DMA,
            pltpu.SemaphoreType.DMA,
            pltpu.SemaphoreType.DMA,
            pltpu.SemaphoreType.DMA,
            pltpu.SemaphoreType.DMA,
            pltpu.SemaphoreType.DMA,
            pltpu.SemaphoreType.DMA,
            pltpu.SemaphoreType.DMA,
        ],
        compiler_params=pltpu.CompilerParams(use_tc_tiling_on_sc=False),
    )(x_flat, token_table, position_table)


def kernel(x, token_table, position_table):
    x_flat = x.reshape(-1).astype(jnp.int32)
    out = _embed(x_flat, token_table, position_table)
    return out.reshape(_BATCH, _SEQ, _HIDDEN)


# speedup vs baseline: 1.3010x; 1.1409x over previous
import functools

import jax
import jax.numpy as jnp
from jax import lax
from jax.experimental import pallas as pl
from jax.experimental.pallas import tpu as pltpu
from jax.experimental.pallas import tpu_sc as plsc

_VOCAB = 1000000
_HIDDEN = 64
_PHYS = 2 * _HIDDEN
_BATCH = 1024
_SEQ = 200

_NC, _NS = 2, 16
_NW = _NC * _NS
_TOTAL = _BATCH * _SEQ
_PER_W = _TOTAL // _NW
_CHUNK = 400
_NCHUNK = _PER_W // _CHUNK


def _gather_body(idx_hbm, tok_hbm, out_hbm,
                 idx0, idx1, rows0, rows1,
                 s_idx0, s_idx1, s_g0, s_g1, s_o0, s_o1):
    wid = lax.axis_index("s") * _NC + lax.axis_index("c")
    base = wid * _PER_W

    idx_v = (idx0, idx1)
    rows_v = (rows0, rows1)
    s_idx = (s_idx0, s_idx1)
    s_g = (s_g0, s_g1)
    s_o = (s_o0, s_o1)

    gather_d = [None, None]
    out_d = [None, None]

    for i in range(_NCHUNK):
        b = i % 2
        off = base + i * _CHUNK
        if i >= 2:
            out_d[b].wait()
        di = pltpu.async_copy(idx_hbm.at[pl.ds(off, _CHUNK)], idx_v[b],
                              s_idx[b])
        di.wait()
        gather_d[b] = pltpu.async_copy(tok_hbm.at[idx_v[b]], rows_v[b],
                                       s_g[b])
        if i >= 1:
            pb = 1 - b
            gather_d[pb].wait()
            out_d[pb] = pltpu.async_copy(
                rows_v[pb],
                out_hbm.at[pl.ds(base + (i - 1) * _CHUNK, _CHUNK)],
                s_o[pb])

    last = (_NCHUNK - 1) % 2
    gather_d[last].wait()
    out_d[last] = pltpu.async_copy(
        rows_v[last],
        out_hbm.at[pl.ds(base + (_NCHUNK - 1) * _CHUNK, _CHUNK)],
        s_o[last])
    out_d[last].wait()
    out_d[1 - last].wait()


_SBLK = 8


def _finish_body(g_ref, pos_ref, out_ref):
    for k in range(_SBLK):
        rows = g_ref[k, :, 0:_HIDDEN]
        out_ref[k] = rows.T + pos_ref[k, :][:, None]


@jax.jit
def _embed(x_flat, tok_pad, pos_t):
    mesh = plsc.VectorSubcoreMesh(core_axis_name="c", subcore_axis_name="s")
    g = pl.kernel(
        _gather_body,
        out_type=jax.ShapeDtypeStruct((_TOTAL, _PHYS), jnp.float32),
        mesh=mesh,
        scratch_types=[
            pltpu.VMEM((_CHUNK,), jnp.int32),
            pltpu.VMEM((_CHUNK,), jnp.int32),
            pltpu.VMEM((_CHUNK, _PHYS), jnp.float32),
            pltpu.VMEM((_CHUNK, _PHYS), jnp.float32),
            pltpu.SemaphoreType.DMA,
            pltpu.SemaphoreType.DMA,
            pltpu.SemaphoreType.DMA,
            pltpu.SemaphoreType.DMA,
            pltpu.SemaphoreType.DMA,
            pltpu.SemaphoreType.DMA,
        ],
        compiler_params=pltpu.CompilerParams(use_tc_tiling_on_sc=True),
    )(x_flat, tok_pad)

    g3 = g.reshape(_SEQ, _BATCH, _PHYS)
    out = pl.pallas_call(
        _finish_body,
        out_shape=jax.ShapeDtypeStruct((_SEQ, _HIDDEN, _BATCH), jnp.float32),
        grid=(_SEQ // _SBLK,),
        in_specs=[
            pl.BlockSpec((_SBLK, _BATCH, _PHYS), lambda s: (s, 0, 0)),
            pl.BlockSpec((_SBLK, _HIDDEN), lambda s: (s, 0)),
        ],
        out_specs=pl.BlockSpec((_SBLK, _HIDDEN, _BATCH), lambda s: (s, 0, 0)),
    )(g3, pos_t)
    return out


def kernel(x, token_table, position_table):
    x_flat = x.T.reshape(-1).astype(jnp.int32)
    tok_pad = jnp.pad(token_table, ((0, 0), (0, _HIDDEN)))
    out = _embed(x_flat, tok_pad, position_table)
    return jnp.transpose(out, (2, 0, 1))


# speedup vs baseline: 1.8238x; 1.4018x over previous
import functools

import jax
import jax.numpy as jnp
from jax import lax
from jax.experimental import pallas as pl
from jax.experimental.pallas import tpu as pltpu
from jax.experimental.pallas import tpu_sc as plsc

_VOCAB = 1000000
_HIDDEN = 64
_BATCH = 1024
_SEQ = 200

_NC, _NS = 2, 16
_NW = _NC * _NS
_TOTAL = _BATCH * _SEQ
_PER_W = _TOTAL // _NW
_CHUNK = 400
_NCHUNK = _PER_W // _CHUNK


def _gather_body(idx_hbm, tok_hbm, out_hbm,
                 idx0, idx1, rows0, rows1,
                 s_idx0, s_idx1, s_row0, s_row1, s_o0, s_o1):
    wid = lax.axis_index("s") * _NC + lax.axis_index("c")
    base = wid * _PER_W

    idx_v = (idx0, idx1)
    rows_v = (rows0, rows1)
    s_idx = (s_idx0, s_idx1)
    s_row = (s_row0, s_row1)
    s_o = (s_o0, s_o1)

    out_d = [None, None]

    def issue(b, off):
        def group(gidx, _):
            v = idx_v[b][pl.ds(gidx * 16, 16)]
            for k in range(16):
                pltpu.async_copy(tok_hbm.at[pl.ds(v[k], 1)],
                                 rows_v[b].at[pl.ds(gidx * 16 + k, 1)],
                                 s_row[b])
            return ()

        lax.fori_loop(0, _CHUNK // 16, group, ())

    def drain(b):
        pltpu.make_async_copy(tok_hbm.at[pl.ds(0, _CHUNK)], rows_v[b],
                              s_row[b]).wait()

    for i in range(_NCHUNK):
        b = i % 2
        off = base + i * _CHUNK
        if i >= 2:
            out_d[b].wait()
        pltpu.async_copy(idx_hbm.at[pl.ds(off, _CHUNK)], idx_v[b],
                         s_idx[b]).wait()
        issue(b, off)
        if i >= 1:
            pb = 1 - b
            drain(pb)
            out_d[pb] = pltpu.async_copy(
                rows_v[pb],
                out_hbm.at[pl.ds(base + (i - 1) * _CHUNK, _CHUNK)],
                s_o[pb])

    last = (_NCHUNK - 1) % 2
    drain(last)
    out_d[last] = pltpu.async_copy(
        rows_v[last],
        out_hbm.at[pl.ds(base + (_NCHUNK - 1) * _CHUNK, _CHUNK)],
        s_o[last])
    out_d[last].wait()
    out_d[1 - last].wait()


_SBLK = 8


def _finish_body(g_ref, pos_ref, out_ref):
    for k in range(_SBLK):
        rows = g_ref[k]
        out_ref[k] = rows.T + pos_ref[k, :][:, None]


@jax.jit
def _embed(x_flat, token_table, pos):
    mesh = plsc.VectorSubcoreMesh(core_axis_name="c", subcore_axis_name="s")
    g = pl.kernel(
        _gather_body,
        out_type=jax.ShapeDtypeStruct((_TOTAL, _HIDDEN), jnp.float32),
        mesh=mesh,
        scratch_types=[
            pltpu.VMEM((_CHUNK,), jnp.int32),
            pltpu.VMEM((_CHUNK,), jnp.int32),
            pltpu.VMEM((_CHUNK, _HIDDEN), jnp.float32),
            pltpu.VMEM((_CHUNK, _HIDDEN), jnp.float32),
            pltpu.SemaphoreType.DMA,
            pltpu.SemaphoreType.DMA,
            pltpu.SemaphoreType.DMA,
            pltpu.SemaphoreType.DMA,
            pltpu.SemaphoreType.DMA,
            pltpu.SemaphoreType.DMA,
        ],
        compiler_params=pltpu.CompilerParams(use_tc_tiling_on_sc=True),
    )(x_flat, token_table)

    g3 = g.reshape(_SEQ, _BATCH, _HIDDEN)
    out = pl.pallas_call(
        _finish_body,
        out_shape=jax.ShapeDtypeStruct((_SEQ, _HIDDEN, _BATCH), jnp.float32),
        grid=(_SEQ // _SBLK,),
        in_specs=[
            pl.BlockSpec((_SBLK, _BATCH, _HIDDEN), lambda s: (s, 0, 0)),
            pl.BlockSpec((_SBLK, _HIDDEN), lambda s: (s, 0)),
        ],
        out_specs=pl.BlockSpec((_SBLK, _HIDDEN, _BATCH), lambda s: (s, 0, 0)),
    )(g3, pos)
    return out


def kernel(x, token_table, position_table):
    x_flat = x.T.reshape(-1).astype(jnp.int32)
    out = _embed(x_flat, token_table, position_table)
    return jnp.transpose(out, (2, 0, 1))
